# baseline (speedup 1.0000x reference)
import functools

import jax
import jax.numpy as jnp
from jax import lax
from jax.experimental import pallas as pl
from jax.experimental.pallas import tpu as pltpu
from jax.experimental.pallas import tpu_sc as plsc

D = 16
NC, NS, LANES = 2, 16, 16
NW = NC * NS
TB = 25

_GATHER_DNUMS = lax.GatherDimensionNumbers(
    offset_dims=(), collapsed_slice_dims=(0,), start_index_map=(0,)
)


def _combo_body(btT, itT, utT, w, bias, c_out):
    code = lax.broadcasted_iota(jnp.int32, (1, 8), 1)
    f1 = (code & 1).astype(jnp.float32)
    f2 = ((code >> 1) & 1).astype(jnp.float32)
    f3 = ((code >> 2) & 1).astype(jnp.float32)
    pb = btT[:, 0:1] + f1 * (btT[:, 1:2] - btT[:, 0:1])
    pi = itT[:, 0:1] + f2 * (itT[:, 1:2] - itT[:, 0:1])
    pu = utT[:, 0:1] + f3 * (utT[:, 1:2] - utT[:, 0:1])
    combT = jnp.concatenate([pb, pi, pu], axis=0)
    ct = jnp.dot(w[...], combT, preferred_element_type=jnp.float32) + bias[...]
    c_out[...] = jnp.concatenate([ct, jnp.zeros((D, 8), jnp.float32)], axis=1)


def _combo_table_t(btT, itT, utT, w, bias2d):
    return pl.pallas_call(
        _combo_body,
        out_shape=jax.ShapeDtypeStruct((D, D), jnp.float32),
    )(btT, itT, utT, w, bias2d)


def _make_sc_lookup(B, T):
    tb = 20
    n_chunk = T // tb
    tt_n = T // 8
    bt_n = B // 128
    mesh = plsc.VectorSubcoreMesh(
        core_axis_name="c", subcore_axis_name="s", num_cores=NC, num_subcores=NS
    )

    ftt = 5
    n_fchunk = tt_n // ftt
    sub_per_f = (ftt * 8) // tb

    @functools.partial(
        pl.kernel,
        mesh=mesh,
        compiler_params=pltpu.CompilerParams(use_tc_tiling_on_sc=False),
        out_type=jax.ShapeDtypeStruct((T, D // 8, bt_n, 8, 128), jnp.float32),
        scratch_types=[
            pltpu.VMEM((2, ftt, 1, 8, 128), jnp.int32),
            pltpu.VMEM((2, ftt, 1, 8, 128), jnp.int32),
            pltpu.VMEM((2, ftt, 1, 8, 128), jnp.int32),
            pltpu.VMEM((2, tb, D // 8, 1, 8, 128), jnp.float32),
            pltpu.VMEM((D, D), jnp.float32),
            pltpu.SemaphoreType.DMA,
            pltpu.SemaphoreType.DMA,
            pltpu.SemaphoreType.DMA,
            pltpu.SemaphoreType.DMA,
        ],
    )
    def sc_lookup(f1_hbm, f2_hbm, f3_hbm, ct_hbm, out_hbm,
                  f1_v, f2_v, f3_v, rows_v, ct_v,
                  sem_in0, sem_in1, sem_out0, sem_out1):
        wid = lax.axis_index("s") * NC + lax.axis_index("c")
        sems_in = (sem_in0, sem_in1)
        sems_out = (sem_out0, sem_out1)
        ct_cp = pltpu.async_copy(ct_hbm, ct_v, sem_in1)

        def start_flags(fi):
            fp = fi % 2
            return [
                pltpu.async_copy(
                    f_hbm.at[pl.ds(fi * ftt, ftt), pl.ds(wid, 1), :, :],
                    f_v.at[fp], sems_in[fp])
                for f_hbm, f_v in ((f1_hbm, f1_v), (f2_hbm, f2_v),
                                   (f3_hbm, f3_v))
            ]

        in_flight = {0: start_flags(0)}
        ct_cp.wait()
        cks = [ct_v[k, :] for k in range(D)]

        out_flight = {}
        for fi in range(n_fchunk):
            fp = fi % 2
            if fi + 1 < n_fchunk:
                in_flight[fi + 1] = start_flags(fi + 1)
            for cp in in_flight.pop(fi):
                cp.wait()
            for sub in range(sub_per_f):
                ci = fi * sub_per_f + sub
                p = ci % 2
                if ci >= 2:
                    out_flight.pop(ci - 2).wait()

                @plsc.parallel_loop(0, tb, 1, unroll=4)
                def t_body(lt, sub=sub, fp=fp, p=p):
                    t = sub * tb + lt
                    tt = t // 8
                    tr = t - tt * 8
                    for g in range(128 // LANES):
                        s = g * LANES
                        a = f1_v[fp, tt, 0, tr, pl.ds(s, LANES)]
                        bb = f2_v[fp, tt, 0, tr, pl.ds(s, LANES)]
                        cc = f3_v[fp, tt, 0, tr, pl.ds(s, LANES)]
                        code16 = a + bb * 2 + cc * 4
                        for k in range(D):
                            outv = lax.gather(
                                cks[k], code16[:, None], _GATHER_DNUMS, (1,),
                                mode=lax.GatherScatterMode.PROMISE_IN_BOUNDS,
                            )
                            rows_v[p, lt, k // 8, 0, k % 8,
                                   pl.ds(s, LANES)] = outv
                out_flight[ci] = pltpu.async_copy(
                    rows_v.at[p],
                    out_hbm.at[pl.ds(ci * tb, tb), :, pl.ds(wid, 1), :, :],
                    sems_out[p],
                )
        for cp in out_flight.values():
            cp.wait()

    return sc_lookup


def kernel(bold_flags, italic_flags, underline_flags,
           bold_table, italic_table, underline_table, W, b):
    B, T = bold_flags.shape
    ct = _combo_table_t(
        bold_table.T, italic_table.T, underline_table.T,
        W, b.reshape(D, 1),
    )
    def native_view(f):
        return (f.astype(jnp.int32)
                .reshape(B // 128, 128, T // 8, 8).transpose(2, 0, 3, 1))

    f1 = native_view(bold_flags)
    f2 = native_view(italic_flags)
    f3 = native_view(underline_flags)
    out5 = _make_sc_lookup(B, T)(f1, f2, f3, ct)
    return out5.transpose((2, 4, 0, 1, 3)).reshape(B, T, D)


# speedup vs baseline: 1.0544x; 1.0544x over previous
import functools

import jax
import jax.numpy as jnp
from jax import lax
from jax.experimental import pallas as pl
from jax.experimental.pallas import tpu as pltpu
from jax.experimental.pallas import tpu_sc as plsc

D = 16
NC, NS, LANES = 2, 16, 16
NW = NC * NS
TB = 25

_GATHER_DNUMS = lax.GatherDimensionNumbers(
    offset_dims=(), collapsed_slice_dims=(0,), start_index_map=(0,)
)


def _combo_body(btT, itT, utT, w, bias, c_out):
    code = lax.broadcasted_iota(jnp.int32, (1, 8), 1)
    f1 = (code & 1).astype(jnp.float32)
    f2 = ((code >> 1) & 1).astype(jnp.float32)
    f3 = ((code >> 2) & 1).astype(jnp.float32)
    pb = btT[:, 0:1] + f1 * (btT[:, 1:2] - btT[:, 0:1])
    pi = itT[:, 0:1] + f2 * (itT[:, 1:2] - itT[:, 0:1])
    pu = utT[:, 0:1] + f3 * (utT[:, 1:2] - utT[:, 0:1])
    combT = jnp.concatenate([pb, pi, pu], axis=0)
    ct = jnp.dot(w[...], combT, preferred_element_type=jnp.float32) + bias[...]
    c_out[...] = jnp.concatenate([ct, jnp.zeros((D, 8), jnp.float32)], axis=1)


def _combo_table_t(btT, itT, utT, w, bias2d):
    return pl.pallas_call(
        _combo_body,
        out_shape=jax.ShapeDtypeStruct((D, D), jnp.float32),
    )(btT, itT, utT, w, bias2d)


def _make_sc_lookup(B, T):
    tb = 20
    n_chunk = T // tb
    tt_n = T // 8
    bt_n = B // 128
    mesh = plsc.VectorSubcoreMesh(
        core_axis_name="c", subcore_axis_name="s", num_cores=NC, num_subcores=NS
    )

    ftt = 5
    n_fchunk = tt_n // ftt
    sub_per_f = (ftt * 8) // tb

    @functools.partial(
        pl.kernel,
        mesh=mesh,
        compiler_params=pltpu.CompilerParams(use_tc_tiling_on_sc=False),
        out_type=jax.ShapeDtypeStruct((T, D // 8, bt_n, 8, 128), jnp.float32),
        scratch_types=[
            pltpu.VMEM((2, ftt, 1, 8, 128), jnp.int32),
            pltpu.VMEM((2, ftt, 1, 8, 128), jnp.int32),
            pltpu.VMEM((2, ftt, 1, 8, 128), jnp.int32),
            pltpu.VMEM((2, tb, D // 8, 1, 8, 128), jnp.float32),
            pltpu.VMEM((D, D), jnp.float32),
            pltpu.SemaphoreType.DMA,
            pltpu.SemaphoreType.DMA,
            pltpu.SemaphoreType.DMA,
            pltpu.SemaphoreType.DMA,
        ],
    )
    def sc_lookup(f1_hbm, f2_hbm, f3_hbm, ct_hbm, out_hbm,
                  f1_v, f2_v, f3_v, rows_v, ct_v,
                  sem_in0, sem_in1, sem_out0, sem_out1):
        wid = lax.axis_index("s") * NC + lax.axis_index("c")
        sems_in = (sem_in0, sem_in1)
        sems_out = (sem_out0, sem_out1)
        ct_cp = pltpu.async_copy(ct_hbm, ct_v, sem_in1)

        def start_flags(fi):
            fp = fi % 2
            return [
                pltpu.async_copy(
                    f_hbm.at[pl.ds(fi * ftt, ftt), pl.ds(wid, 1), :, :],
                    f_v.at[fp], sems_in[fp])
                for f_hbm, f_v in ((f1_hbm, f1_v), (f2_hbm, f2_v),
                                   (f3_hbm, f3_v))
            ]

        in_flight = {0: start_flags(0)}
        ct_cp.wait()
        cks = [ct_v[k, :] for k in range(D)]

        out_flight = {}
        for fi in range(n_fchunk):
            fp = fi % 2
            if fi + 1 < n_fchunk:
                in_flight[fi + 1] = start_flags(fi + 1)
            for cp in in_flight.pop(fi):
                cp.wait()
            for sub in range(sub_per_f):
                ci = fi * sub_per_f + sub
                p = ci % 2
                if ci >= 2:
                    out_flight.pop(ci - 2).wait()

                @plsc.parallel_loop(0, tb, 1, unroll=2)
                def t_body(lt, sub=sub, fp=fp, p=p):
                    t = sub * tb + lt
                    tt = t // 8
                    tr = t - tt * 8
                    for g in range(128 // LANES):
                        s = g * LANES
                        a = f1_v[fp, tt, 0, tr, pl.ds(s, LANES)]
                        bb = f2_v[fp, tt, 0, tr, pl.ds(s, LANES)]
                        cc = f3_v[fp, tt, 0, tr, pl.ds(s, LANES)]
                        code16 = a + bb * 2 + cc * 4
                        for k in range(D):
                            outv = lax.gather(
                                cks[k], code16[:, None], _GATHER_DNUMS, (1,),
                                mode=lax.GatherScatterMode.PROMISE_IN_BOUNDS,
                            )
                            rows_v[p, lt, k // 8, 0, k % 8,
                                   pl.ds(s, LANES)] = outv
                out_flight[ci] = pltpu.async_copy(
                    rows_v.at[p],
                    out_hbm.at[pl.ds(ci * tb, tb), :, pl.ds(wid, 1), :, :],
                    sems_out[p],
                )
        for cp in out_flight.values():
            cp.wait()

    return sc_lookup


def kernel(bold_flags, italic_flags, underline_flags,
           bold_table, italic_table, underline_table, W, b):
    B, T = bold_flags.shape
    ct = _combo_table_t(
        bold_table.T, italic_table.T, underline_table.T,
        W, b.reshape(D, 1),
    )
    def native_view(f):
        return (f.astype(jnp.int32)
                .reshape(B // 128, 128, T // 8, 8).transpose(2, 0, 3, 1))

    f1 = native_view(bold_flags)
    f2 = native_view(italic_flags)
    f3 = native_view(underline_flags)
    out5 = _make_sc_lookup(B, T)(f1, f2, f3, ct)
    return out5.transpose((2, 4, 0, 1, 3)).reshape(B, T, D)
